# baseline (speedup 1.0000x reference)
import jax
import jax.numpy as jnp
from jax.experimental import pallas as pl
from jax.experimental.pallas import tpu as pltpu

PT, PH, PW = 1, 5, 5
D = PT * PH * PW
DP = 32
NQ = 16928
M_TILE = 4352
K_TILE = 1024
NP = 17408
BIG = 1e30


def _unfold_t(video):
    T, H, W = video.shape[2], video.shape[3], video.shape[4]
    To, Ho, Wo = T - PT + 1, H - PH + 1, W - PW + 1
    slices = []
    for dt in range(PT):
        for dh in range(PH):
            for dw in range(PW):
                slices.append(video[0, 0, dt:dt + To, dh:dh + Ho, dw:dw + Wo])
    p = jnp.stack(slices, axis=0)
    return p.reshape(D, To * Ho * Wo)


def _nn_loss_body(q_ref, kt_ref, out_ref):
    i = pl.program_id(0)
    q = q_ref[...]

    def chunk(j):
        kt = kt_ref[:, j * K_TILE:(j + 1) * K_TILE]
        return jax.lax.dot_general(
            q, kt, (((0,), (0,)), ((), ())),
            preferred_element_type=jnp.float32)

    def col_tree(part):
        red = part[:, 0:128]
        for c in range(1, K_TILE // 128):
            red = jnp.minimum(red, part[:, c * 128:(c + 1) * 128])
        return red

    carry = col_tree(chunk(0))
    for j in range(1, NP // K_TILE):
        carry = jnp.minimum(carry, col_tree(chunk(j)))
    mins = jnp.min(carry, axis=1, keepdims=True)

    qf = q.astype(jnp.float32)
    n_real = jnp.minimum(NQ - i * M_TILE, M_TILE).astype(jnp.float32)
    tile_sum = jnp.sum(qf * qf) - 2.0 * n_real + jnp.sum(mins)

    @pl.when(i == 0)
    def _init():
        out_ref[...] = jnp.zeros_like(out_ref)

    out_ref[...] += jnp.full((8, 128), tile_sum, jnp.float32)


def kernel(result, valid_video):
    qt = _unfold_t(result)
    kt = _unfold_t(valid_video)

    ones = jnp.ones((2, NQ), jnp.float32)
    qa = jnp.pad(
        jnp.concatenate([qt, ones], axis=0).astype(jnp.bfloat16),
        ((0, DP - D - 2), (0, NP - NQ)))

    ktp = jnp.pad(kt, ((0, 0), (0, NP - NQ)))
    ksq = jnp.concatenate(
        [jnp.sum(kt * kt, axis=0), jnp.full((NP - NQ,), BIG, jnp.float32)])
    ksq_hi = ksq.astype(jnp.bfloat16).astype(jnp.float32)
    ksq_lo = ksq - ksq_hi
    ka = jnp.pad(
        jnp.concatenate(
            [-2.0 * ktp, ksq_hi[None, :], ksq_lo[None, :]],
            axis=0).astype(jnp.bfloat16),
        ((0, DP - D - 2), (0, 0)))

    out = pl.pallas_call(
        _nn_loss_body,
        grid=(NP // M_TILE,),
        in_specs=[
            pl.BlockSpec((DP, M_TILE), lambda i: (0, i)),
            pl.BlockSpec((DP, NP), lambda i: (0, 0)),
        ],
        out_specs=pl.BlockSpec((8, 128), lambda i: (0, 0)),
        out_shape=jax.ShapeDtypeStruct((8, 128), jnp.float32),
        compiler_params=pltpu.CompilerParams(
            dimension_semantics=("arbitrary",)),
    )(qa, ka)
    return out[0, 0]


# speedup vs baseline: 1.3057x; 1.3057x over previous
import jax
import jax.numpy as jnp
from jax.experimental import pallas as pl
from jax.experimental.pallas import tpu as pltpu

PT, PH, PW = 1, 5, 5
D = PT * PH * PW
DP = 32
NQ = 16928
M_TILE = 2176
K_TILE = 1024
NP = 17408
BIG = 1e30


def _unfold_t(video):
    T, H, W = video.shape[2], video.shape[3], video.shape[4]
    To, Ho, Wo = T - PT + 1, H - PH + 1, W - PW + 1
    slices = []
    for dt in range(PT):
        for dh in range(PH):
            for dw in range(PW):
                slices.append(video[0, 0, dt:dt + To, dh:dh + Ho, dw:dw + Wo])
    p = jnp.stack(slices, axis=0)
    return p.reshape(D, To * Ho * Wo)


def _nn_loss_body(q_ref, kt_ref, out_ref):
    i = pl.program_id(0)
    q = q_ref[...]

    def chunk(j):
        kt = kt_ref[:, j * K_TILE:(j + 1) * K_TILE]
        return jax.lax.dot_general(
            q, kt, (((0,), (0,)), ((), ())),
            preferred_element_type=jnp.float32)

    def col_tree(part):
        red = part[:, 0:128]
        for c in range(1, K_TILE // 128):
            red = jnp.minimum(red, part[:, c * 128:(c + 1) * 128])
        return red

    carry = col_tree(chunk(0))
    for j in range(1, NP // K_TILE):
        carry = jnp.minimum(carry, col_tree(chunk(j)))
    mins = jnp.min(carry, axis=1, keepdims=True)

    qf = q.astype(jnp.float32)
    n_real = jnp.minimum(NQ - i * M_TILE, M_TILE).astype(jnp.float32)
    tile_sum = jnp.sum(qf * qf) - 2.0 * n_real + jnp.sum(mins)

    @pl.when(i == 0)
    def _init():
        out_ref[...] = jnp.zeros_like(out_ref)

    out_ref[...] += jnp.full((8, 128), tile_sum, jnp.float32)


def kernel(result, valid_video):
    qt = _unfold_t(result.astype(jnp.bfloat16))
    kt = _unfold_t(valid_video.astype(jnp.bfloat16))

    ones = jnp.ones((2, NQ), jnp.bfloat16)
    qa = jnp.pad(
        jnp.concatenate([qt, ones], axis=0),
        ((0, DP - D - 2), (0, NP - NQ)))

    ktf = kt.astype(jnp.float32)
    ktp = jnp.pad(kt * jnp.bfloat16(-2.0), ((0, 0), (0, NP - NQ)))
    ksq = jnp.concatenate(
        [jnp.sum(ktf * ktf, axis=0), jnp.full((NP - NQ,), BIG, jnp.float32)])
    ksq_hi = ksq.astype(jnp.bfloat16).astype(jnp.float32)
    ksq_lo = (ksq - ksq_hi).astype(jnp.bfloat16)
    ka = jnp.pad(
        jnp.concatenate(
            [ktp, ksq_hi.astype(jnp.bfloat16)[None, :], ksq_lo[None, :]],
            axis=0),
        ((0, DP - D - 2), (0, 0)))

    out = pl.pallas_call(
        _nn_loss_body,
        grid=(NP // M_TILE,),
        in_specs=[
            pl.BlockSpec((DP, M_TILE), lambda i: (0, i)),
            pl.BlockSpec((DP, NP), lambda i: (0, 0)),
        ],
        out_specs=pl.BlockSpec((8, 128), lambda i: (0, 0)),
        out_shape=jax.ShapeDtypeStruct((8, 128), jnp.float32),
        compiler_params=pltpu.CompilerParams(
            dimension_semantics=("arbitrary",)),
    )(qa, ka)
    return out[0, 0]
